# baseline (speedup 1.0000x reference)
import functools

import jax
import jax.numpy as jnp
from jax.experimental import pallas as pl
from jax.experimental.pallas import tpu as pltpu

K_TOP = 64


def _encode_kernel(x_ref, w_ref, b_ref, out_ref):
    out_ref[...] = (
        jnp.dot(x_ref[...], w_ref[...], preferred_element_type=jnp.float32)
        + b_ref[...]
    )


def _bf16_kth_bisect(data_bf, k):
    rows = data_bf.shape[0]
    tb = jnp.zeros((rows, 1), jnp.int32)
    for b in range(30, 15, -1):
        cand = tb + jnp.int32(1 << b)
        tf = jax.lax.bitcast_convert_type(cand, jnp.float32).astype(jnp.bfloat16)
        cnt = jnp.sum((data_bf >= tf).astype(jnp.float32), axis=1, keepdims=True)
        tb = jnp.where(cnt >= float(k), cand, tb)
    return tb


def _threshold_kernel(p_ref, thr_ref, *, k):
    p = p_ref[...]
    cpos = jnp.sum((p > 0.0).astype(jnp.float32), axis=1, keepdims=True)
    pos_ok = cpos >= float(k)

    pb = p.astype(jnp.bfloat16)
    tb = _bf16_kth_bisect(pb, k)
    base = jax.lax.bitcast_convert_type(
        jnp.maximum(tb - jnp.int32(0x10000), 0), jnp.float32
    )
    r2 = (p - base).astype(jnp.bfloat16)
    tb2 = _bf16_kth_bisect(r2, k)
    b2 = jax.lax.bitcast_convert_type(tb2, jnp.float32).astype(jnp.bfloat16)

    psel = jnp.where(r2 == b2, p, -jnp.inf)
    cgt2 = jnp.sum((r2 > b2).astype(jnp.float32), axis=1, keepdims=True)
    rcnt = jnp.where(pos_ok, float(k) - cgt2, 0.0)
    u0 = jnp.full_like(cgt2, jnp.inf)

    def cond(carry):
        _, rc = carry
        return jnp.any(rc > 0.0)

    def body(carry):
        u, rc = carry
        masked = jnp.where(psel < u, psel, -jnp.inf)
        v = jnp.max(masked, axis=1, keepdims=True)
        mult = jnp.sum((masked == v).astype(jnp.float32), axis=1, keepdims=True)
        active = rc > 0.0
        u = jnp.where(active, v, u)
        rc = rc - jnp.where(active, mult, 0.0)
        return u, rc

    u, _ = jax.lax.while_loop(cond, body, (u0, rcnt))
    thr_ref[...] = jnp.where(pos_ok, u, 0.0)


def _decode_kernel(p_ref, thr_ref, w_ref, b_ref, lat_ref, out_ref):
    lt = pl.program_id(1)
    p = p_ref[...]
    lat = jnp.where(p >= thr_ref[...], jnp.maximum(p, 0.0), 0.0)
    lat_ref[...] = lat
    contrib = jnp.dot(lat, w_ref[...], preferred_element_type=jnp.float32)

    @pl.when(lt == 0)
    def _():
        out_ref[...] = contrib + b_ref[...]

    @pl.when(lt != 0)
    def _():
        out_ref[...] += contrib


@jax.jit
def kernel(x, W_enc, b_enc, W_dec, b_dec):
    n, d = x.shape
    l = W_enc.shape[1]

    r1 = min(512, n)
    lt_size = min(2048, l)
    n_lt = l // lt_size
    n_nb = n // r1

    b_enc2 = b_enc.reshape(1, l)
    b_dec2 = b_dec.reshape(1, d)

    preact = pl.pallas_call(
        _encode_kernel,
        grid=(n_lt, n_nb),
        in_specs=[
            pl.BlockSpec((r1, d), lambda lt, nb: (nb, 0)),
            pl.BlockSpec((d, lt_size), lambda lt, nb: (0, lt)),
            pl.BlockSpec((1, lt_size), lambda lt, nb: (0, lt)),
        ],
        out_specs=pl.BlockSpec((r1, lt_size), lambda lt, nb: (nb, lt)),
        out_shape=jax.ShapeDtypeStruct((n, l), jnp.float32),
        compiler_params=pltpu.CompilerParams(
            dimension_semantics=("arbitrary", "arbitrary"),
        ),
    )(x, W_enc, b_enc2)

    r_thr = min(64, n)
    thresholds = pl.pallas_call(
        functools.partial(_threshold_kernel, k=K_TOP),
        grid=(n // r_thr,),
        in_specs=[pl.BlockSpec((r_thr, l), lambda i: (i, 0))],
        out_specs=pl.BlockSpec((r_thr, 1), lambda i: (i, 0)),
        out_shape=jax.ShapeDtypeStruct((n, 1), jnp.float32),
    )(preact)

    r2 = min(1024, n)
    latents, out = pl.pallas_call(
        _decode_kernel,
        grid=(n // r2, n_lt),
        in_specs=[
            pl.BlockSpec((r2, lt_size), lambda nb, lt: (nb, lt)),
            pl.BlockSpec((r2, 1), lambda nb, lt: (nb, 0)),
            pl.BlockSpec((lt_size, d), lambda nb, lt: (lt, 0)),
            pl.BlockSpec((1, d), lambda nb, lt: (0, 0)),
        ],
        out_specs=[
            pl.BlockSpec((r2, lt_size), lambda nb, lt: (nb, lt)),
            pl.BlockSpec((r2, d), lambda nb, lt: (nb, 0)),
        ],
        out_shape=[
            jax.ShapeDtypeStruct((n, l), jnp.float32),
            jax.ShapeDtypeStruct((n, d), jnp.float32),
        ],
        compiler_params=pltpu.CompilerParams(
            dimension_semantics=("parallel", "arbitrary"),
        ),
    )(preact, thresholds, W_dec, b_dec2)

    num_dead = jnp.array(0, dtype=jnp.int32)
    return (latents, out, preact, num_dead)


# speedup vs baseline: 2.3541x; 2.3541x over previous
import functools

import jax
import jax.numpy as jnp
from jax.experimental import pallas as pl
from jax.experimental.pallas import tpu as pltpu

K_TOP = 64


def _encode_kernel(x_ref, w_ref, b_ref, out_ref):
    out_ref[...] = (
        jnp.dot(x_ref[...], w_ref[...], preferred_element_type=jnp.float32)
        + b_ref[...]
    )


def _threshold_kernel(p_ref, thr_ref, *, k):
    p = p_ref[...]
    rows, width = p.shape
    kf = float(k)

    m = jnp.max(p.reshape(rows, 8, width // 8), axis=1)

    mum = jnp.mean(m, axis=1, keepdims=True)
    msq = jnp.mean(m * m, axis=1, keepdims=True)
    sigm = jnp.sqrt(jnp.maximum(msq - mum * mum, 1e-30))

    t0 = mum + 2.2 * sigm
    lo0 = jnp.zeros((rows, 1), jnp.int32)
    hi0 = jnp.full((rows, 1), jnp.int32(0x7F800000))
    tbest0 = jnp.zeros((rows, 1), jnp.float32)
    found0 = jnp.zeros((rows, 1), jnp.float32)

    def pooled_cond(carry):
        return jnp.any(carry[4] == 0.0)

    def pooled_body(carry):
        t, lo, hi, tbest, found = carry
        cnt = jnp.sum((m >= t).astype(jnp.float32), axis=1, keepdims=True)
        hit = (found == 0.0) & (cnt == kf)
        tbest = jnp.where(hit, t, tbest)
        found = jnp.where(hit, 1.0, found)
        live = found == 0.0
        tkey = jax.lax.bitcast_convert_type(t, jnp.int32)
        lo = jnp.where(live & (cnt > kf), jnp.maximum(lo, tkey), lo)
        hi = jnp.where(live & (cnt < kf), jnp.minimum(hi, tkey), hi)
        collapse = live & (hi - lo <= 1)
        tbest = jnp.where(
            collapse, jax.lax.bitcast_convert_type(lo, jnp.float32), tbest
        )
        found = jnp.where(collapse, 1.0, found)
        arg = (t - mum) * (t - mum) + 2.0 * sigm * sigm * jnp.log(
            jnp.maximum(cnt, 0.5) / kf
        )
        tn = mum + jnp.sqrt(jnp.maximum(arg, 0.0))
        kn = jax.lax.bitcast_convert_type(tn, jnp.int32)
        bad = (kn <= lo) | (kn >= hi)
        kmid = lo + ((hi - lo) >> 1)
        kn = jnp.where(bad, kmid, kn)
        t = jnp.where(
            found == 0.0, jax.lax.bitcast_convert_type(kn, jnp.float32), t
        )
        return t, lo, hi, tbest, found

    _, _, _, tbest, _ = jax.lax.while_loop(
        pooled_cond, pooled_body, (t0, lo0, hi0, tbest0, found0)
    )

    cfull = jnp.sum((p >= tbest).astype(jnp.float32), axis=1, keepdims=True)

    def rem_cond(carry):
        return jnp.any(carry[1] > kf)

    def rem_body(carry):
        t, c = carry
        v = jnp.min(jnp.where(p >= t, p, jnp.inf), axis=1, keepdims=True)
        v = jnp.abs(v)
        mult = jnp.sum((p == v).astype(jnp.float32), axis=1, keepdims=True)
        candc = c - mult
        ok = candc >= kf
        nxt = jax.lax.bitcast_convert_type(
            jax.lax.bitcast_convert_type(v, jnp.int32) + 1, jnp.float32
        )
        active = c > kf
        t = jnp.where(active, jnp.where(ok, nxt, v), t)
        c = jnp.where(active, jnp.where(ok, candc, kf), c)
        return t, c

    t_fin, _ = jax.lax.while_loop(rem_cond, rem_body, (tbest, cfull))
    thr_ref[...] = t_fin


def _decode_kernel(p_ref, thr_ref, w_ref, b_ref, lat_ref, out_ref):
    lt = pl.program_id(1)
    p = p_ref[...]
    lat = jnp.where(p >= thr_ref[...], jnp.maximum(p, 0.0), 0.0)
    lat_ref[...] = lat
    contrib = jnp.dot(lat, w_ref[...], preferred_element_type=jnp.float32)

    @pl.when(lt == 0)
    def _():
        out_ref[...] = contrib + b_ref[...]

    @pl.when(lt != 0)
    def _():
        out_ref[...] += contrib


@jax.jit
def kernel(x, W_enc, b_enc, W_dec, b_dec):
    n, d = x.shape
    l = W_enc.shape[1]

    r1 = min(512, n)
    lt_size = min(2048, l)
    n_lt = l // lt_size
    n_nb = n // r1

    b_enc2 = b_enc.reshape(1, l)
    b_dec2 = b_dec.reshape(1, d)

    preact = pl.pallas_call(
        _encode_kernel,
        grid=(n_lt, n_nb),
        in_specs=[
            pl.BlockSpec((r1, d), lambda lt, nb: (nb, 0)),
            pl.BlockSpec((d, lt_size), lambda lt, nb: (0, lt)),
            pl.BlockSpec((1, lt_size), lambda lt, nb: (0, lt)),
        ],
        out_specs=pl.BlockSpec((r1, lt_size), lambda lt, nb: (nb, lt)),
        out_shape=jax.ShapeDtypeStruct((n, l), jnp.float32),
        compiler_params=pltpu.CompilerParams(
            dimension_semantics=("arbitrary", "arbitrary"),
        ),
    )(x, W_enc, b_enc2)

    r_thr = min(64, n)
    thresholds = pl.pallas_call(
        functools.partial(_threshold_kernel, k=K_TOP),
        grid=(n // r_thr,),
        in_specs=[pl.BlockSpec((r_thr, l), lambda i: (i, 0))],
        out_specs=pl.BlockSpec((r_thr, 1), lambda i: (i, 0)),
        out_shape=jax.ShapeDtypeStruct((n, 1), jnp.float32),
    )(preact)

    r2 = min(1024, n)
    latents, out = pl.pallas_call(
        _decode_kernel,
        grid=(n // r2, n_lt),
        in_specs=[
            pl.BlockSpec((r2, lt_size), lambda nb, lt: (nb, lt)),
            pl.BlockSpec((r2, 1), lambda nb, lt: (nb, 0)),
            pl.BlockSpec((lt_size, d), lambda nb, lt: (lt, 0)),
            pl.BlockSpec((1, d), lambda nb, lt: (0, 0)),
        ],
        out_specs=[
            pl.BlockSpec((r2, lt_size), lambda nb, lt: (nb, lt)),
            pl.BlockSpec((r2, d), lambda nb, lt: (nb, 0)),
        ],
        out_shape=[
            jax.ShapeDtypeStruct((n, l), jnp.float32),
            jax.ShapeDtypeStruct((n, d), jnp.float32),
        ],
        compiler_params=pltpu.CompilerParams(
            dimension_semantics=("parallel", "arbitrary"),
        ),
    )(preact, thresholds, W_dec, b_dec2)

    num_dead = jnp.array(0, dtype=jnp.int32)
    return (latents, out, preact, num_dead)
